# Initial kernel scaffold; baseline (speedup 1.0000x reference)
import jax
import jax.numpy as jnp
from jax.experimental import pallas as pl


def kernel(node_type, table):
    raise NotImplementedError("write your pallas kernel here")


# speedup vs baseline: 1.4487x; 1.4487x over previous
import jax
import jax.numpy as jnp
from jax import lax
from jax.experimental import pallas as pl
from jax.experimental.pallas import tpu as pltpu
from jax.experimental.pallas import tpu_sc as plsc

N_NODES = 100000
DIM = 128
NUM_WORKERS = 32
CHUNK = 128
FULL_CHUNKS = N_NODES // CHUNK
TAIL = N_NODES - FULL_CHUNKS * CHUNK
TOTAL_CHUNKS = FULL_CHUNKS + 1
ITERS = -(-TOTAL_CHUNKS // NUM_WORKERS)


def _embed_body(idx_hbm, table_hbm, out_hbm, idx_v, rows_v, idx_t, rows_t, sem):
    wid = lax.axis_index("s") * 2 + lax.axis_index("c")

    def body(j, carry):
        c = wid + j * NUM_WORKERS

        @pl.when(c < FULL_CHUNKS)
        def _():
            base = c * CHUNK
            pltpu.sync_copy(idx_hbm.at[pl.ds(base, CHUNK)], idx_v)
            pltpu.async_copy(table_hbm.at[idx_v], rows_v, sem).wait()
            pltpu.sync_copy(rows_v, out_hbm.at[pl.ds(base, CHUNK)])

        @pl.when(c == FULL_CHUNKS)
        def _():
            base = FULL_CHUNKS * CHUNK
            pltpu.sync_copy(idx_hbm.at[pl.ds(base, TAIL)], idx_t)
            pltpu.async_copy(table_hbm.at[idx_t], rows_t, sem).wait()
            pltpu.sync_copy(rows_t, out_hbm.at[pl.ds(base, TAIL)])

        return carry

    lax.fori_loop(0, ITERS, body, 0)


def kernel(node_type, table):
    idx = node_type.astype(jnp.int32)
    mesh = plsc.VectorSubcoreMesh(core_axis_name="c", subcore_axis_name="s")
    f = pl.kernel(
        _embed_body,
        mesh=mesh,
        out_type=jax.ShapeDtypeStruct((N_NODES, DIM), jnp.float32),
        scratch_types=[
            pltpu.VMEM((CHUNK,), jnp.int32),
            pltpu.VMEM((CHUNK, DIM), jnp.float32),
            pltpu.VMEM((TAIL,), jnp.int32),
            pltpu.VMEM((TAIL, DIM), jnp.float32),
            pltpu.SemaphoreType.DMA,
        ],
    )
    return f(idx, table)


# speedup vs baseline: 5.5279x; 3.8158x over previous
import jax
import jax.numpy as jnp
from jax import lax
from jax.experimental import pallas as pl
from jax.experimental.pallas import tpu as pltpu
from jax.experimental.pallas import tpu_sc as plsc

N_NODES = 100000
TYPES = 100
DIM = 128
NW = 32
CHUNK = 128
SLOTS = 25
PAD = NW * SLOTS * CHUNK
FULL_CHUNKS = N_NODES // CHUNK
TAIL = N_NODES - FULL_CHUNKS * CHUNK
NBUF = 4


def _body(idx_hbm, table_hbm, out_hbm, table_v, idx_v,
          b0, b1, b2, b3, gsem, s0, s1, s2, s3):
    wid = lax.axis_index("s") * 2 + lax.axis_index("c")
    bufs = (b0, b1, b2, b3)
    sems = (s0, s1, s2, s3)

    @pl.when(lax.axis_index("s") == 0)
    def _stage_table():
        pltpu.sync_copy(table_hbm, table_v)
    plsc.subcore_barrier()
    pltpu.sync_copy(idx_hbm.at[wid], idx_v)

    def out_base(j):
        return (wid * SLOTS + j) * CHUNK

    def gather(j, b):
        pltpu.async_copy(table_v.at[idx_v.at[j]], bufs[b], gsem).wait()

    def scat_issue(j, b):
        pltpu.async_copy(bufs[b], out_hbm.at[pl.ds(out_base(j), CHUNK)], sems[b])

    def scat_wait(j, b):
        pltpu.make_async_copy(
            bufs[b], out_hbm.at[pl.ds(out_base(j), CHUNK)], sems[b]).wait()

    @pl.when(wid < NW - 1)
    def _main():
        for b in range(NBUF):
            gather(b, b)
            scat_issue(b, b)

        def grp(i, carry):
            for b in range(NBUF):
                j = i * NBUF + b
                scat_wait(j - NBUF, b)
                gather(j, b)
                scat_issue(j, b)
            return carry

        lax.fori_loop(1, SLOTS // NBUF, grp, 0)

        scat_wait(SLOTS - 1 - NBUF, 0)
        gather(SLOTS - 1, 0)
        scat_issue(SLOTS - 1, 0)

        scat_wait(SLOTS - 1, 0)
        scat_wait(21, 1)
        scat_wait(22, 2)
        scat_wait(23, 3)

    @pl.when(wid == NW - 1)
    def _last():
        for j in range(6):
            b = j % 2
            gather(j, b)
            pltpu.async_copy(
                bufs[b], out_hbm.at[pl.ds(out_base(j), CHUNK)], sems[b]).wait()
        gather(6, 2)
        pltpu.async_copy(
            b2.at[pl.ds(0, TAIL)],
            out_hbm.at[pl.ds(FULL_CHUNKS * CHUNK, TAIL)], s2).wait()


def kernel(node_type, table):
    idx = jnp.pad(node_type.astype(jnp.int32),
                  (0, PAD - N_NODES)).reshape(NW, SLOTS, CHUNK)
    mesh = plsc.VectorSubcoreMesh(core_axis_name="c", subcore_axis_name="s")
    f = pl.kernel(
        _body,
        mesh=mesh,
        out_type=jax.ShapeDtypeStruct((N_NODES, DIM), jnp.float32),
        scratch_types=[
            pltpu.VMEM_SHARED((TYPES, DIM), jnp.float32),
            pltpu.VMEM((SLOTS, CHUNK), jnp.int32),
            *[pltpu.VMEM((CHUNK, DIM), jnp.float32) for _ in range(NBUF)],
            pltpu.SemaphoreType.DMA,
            *[pltpu.SemaphoreType.DMA for _ in range(NBUF)],
        ],
    )
    return f(idx, table)


# speedup vs baseline: 5.7728x; 1.0443x over previous
import jax
import jax.numpy as jnp
from jax import lax
from jax.experimental import pallas as pl
from jax.experimental.pallas import tpu as pltpu
from jax.experimental.pallas import tpu_sc as plsc

N_NODES = 100000
TYPES = 100
DIM = 128
NW = 32
CHUNK = 128
SLOTS = 25
PAD = NW * SLOTS * CHUNK
FULL_CHUNKS = N_NODES // CHUNK
TAIL = N_NODES - FULL_CHUNKS * CHUNK
NBUF = 4


def _body(idx_hbm, table_hbm, out_hbm, table_v, idx_v,
          b0, b1, b2, b3, gsem, s0, s1, s2, s3):
    wid = lax.axis_index("s") * 2 + lax.axis_index("c")
    bufs = (b0, b1, b2, b3)
    sems = (s0, s1, s2, s3)

    @pl.when(lax.axis_index("s") == 0)
    def _stage_table():
        pltpu.sync_copy(table_hbm, table_v)
    plsc.subcore_barrier()
    pltpu.sync_copy(idx_hbm.at[wid], idx_v)

    def out_base(j):
        return (wid * SLOTS + j) * CHUNK

    def gather_issue(j, b):
        pltpu.async_copy(table_v.at[idx_v.at[j]], bufs[b], gsem)

    def gather_wait(j, b):
        pltpu.make_async_copy(table_v.at[idx_v.at[j]], bufs[b], gsem).wait()

    def gather(j, b):
        gather_issue(j, b)
        gather_wait(j, b)

    def scat_issue(j, b):
        pltpu.async_copy(bufs[b], out_hbm.at[pl.ds(out_base(j), CHUNK)], sems[b])

    def scat_wait(j, b):
        pltpu.make_async_copy(
            bufs[b], out_hbm.at[pl.ds(out_base(j), CHUNK)], sems[b]).wait()

    @pl.when(wid < NW - 1)
    def _main():
        gather_issue(0, 0)
        for j in range(3):
            gather_issue(j + 1, j + 1)
            gather_wait(j, j)
            scat_issue(j, j)

        def steady(i, carry):
            for b in range(NBUF):
                j = i * NBUF + b
                nb = (b + 1) % NBUF
                scat_wait(j - 3, nb)
                gather_issue(j + 1, nb)
                gather_wait(j, b)
                scat_issue(j, b)
            return carry

        scat_wait(0, 0)
        gather_issue(4, 0)
        gather_wait(3, 3)
        scat_issue(3, 3)

        lax.fori_loop(1, SLOTS // NBUF, steady, 0)

        gather_wait(SLOTS - 1, 0)
        scat_issue(SLOTS - 1, 0)

        scat_wait(21, 1)
        scat_wait(22, 2)
        scat_wait(23, 3)
        scat_wait(SLOTS - 1, 0)

    @pl.when(wid == NW - 1)
    def _last():
        for j in range(6):
            b = j % 2
            gather(j, b)
            pltpu.async_copy(
                bufs[b], out_hbm.at[pl.ds(out_base(j), CHUNK)], sems[b]).wait()
        gather(6, 2)
        pltpu.async_copy(
            b2.at[pl.ds(0, TAIL)],
            out_hbm.at[pl.ds(FULL_CHUNKS * CHUNK, TAIL)], s2).wait()


def kernel(node_type, table):
    idx = jnp.pad(node_type.astype(jnp.int32),
                  (0, PAD - N_NODES)).reshape(NW, SLOTS, CHUNK)
    mesh = plsc.VectorSubcoreMesh(core_axis_name="c", subcore_axis_name="s")
    f = pl.kernel(
        _body,
        mesh=mesh,
        out_type=jax.ShapeDtypeStruct((N_NODES, DIM), jnp.float32),
        scratch_types=[
            pltpu.VMEM_SHARED((TYPES, DIM), jnp.float32),
            pltpu.VMEM((SLOTS, CHUNK), jnp.int32),
            *[pltpu.VMEM((CHUNK, DIM), jnp.float32) for _ in range(NBUF)],
            pltpu.SemaphoreType.DMA,
            *[pltpu.SemaphoreType.DMA for _ in range(NBUF)],
        ],
    )
    return f(idx, table)


# speedup vs baseline: 5.7737x; 1.0002x over previous
import jax
import jax.numpy as jnp
from jax import lax
from jax.experimental import pallas as pl
from jax.experimental.pallas import tpu as pltpu
from jax.experimental.pallas import tpu_sc as plsc

N_NODES = 100000
TYPES = 100
DIM = 128
NW = 32
CHUNK = 128
SLOTS = 25
PER_W = SLOTS * CHUNK
LAST_BASE = (NW - 1) * PER_W
LAST_N = N_NODES - LAST_BASE
LAST_SLOTS = LAST_N // CHUNK
TAIL = LAST_N - LAST_SLOTS * CHUNK
NBUF = 4


def _body(idx_hbm, table_hbm, out_hbm, table_v, idx_v,
          b0, b1, b2, b3, gsem, s0, s1, s2, s3):
    wid = lax.axis_index("s") * 2 + lax.axis_index("c")
    bufs = (b0, b1, b2, b3)
    sems = (s0, s1, s2, s3)

    @pl.when(lax.axis_index("s") == 0)
    def _stage_table():
        pltpu.sync_copy(table_hbm, table_v)
    plsc.subcore_barrier()

    base = wid * PER_W

    def out_base(j):
        return base + j * CHUNK

    def gather_issue(j, b):
        pltpu.async_copy(
            table_v.at[idx_v.at[pl.ds(j * CHUNK, CHUNK)]], bufs[b], gsem)

    def gather_wait(j, b):
        pltpu.make_async_copy(
            table_v.at[idx_v.at[pl.ds(j * CHUNK, CHUNK)]], bufs[b], gsem).wait()

    def scat_issue(j, b):
        pltpu.async_copy(bufs[b], out_hbm.at[pl.ds(out_base(j), CHUNK)], sems[b])

    def scat_wait(j, b):
        pltpu.make_async_copy(
            bufs[b], out_hbm.at[pl.ds(out_base(j), CHUNK)], sems[b]).wait()

    @pl.when(wid < NW - 1)
    def _main():
        pltpu.sync_copy(idx_hbm.at[pl.ds(base, PER_W)], idx_v)
        gather_issue(0, 0)
        for j in range(3):
            gather_issue(j + 1, j + 1)
            gather_wait(j, j)
            scat_issue(j, j)

        def steady(i, carry):
            for b in range(NBUF):
                j = i * NBUF + b
                nb = (b + 1) % NBUF
                scat_wait(j - 3, nb)
                gather_issue(j + 1, nb)
                gather_wait(j, b)
                scat_issue(j, b)
            return carry

        scat_wait(0, 0)
        gather_issue(4, 0)
        gather_wait(3, 3)
        scat_issue(3, 3)

        lax.fori_loop(1, SLOTS // NBUF, steady, 0)

        gather_wait(SLOTS - 1, 0)
        scat_issue(SLOTS - 1, 0)

        scat_wait(21, 1)
        scat_wait(22, 2)
        scat_wait(23, 3)
        scat_wait(SLOTS - 1, 0)

    @pl.when(wid == NW - 1)
    def _last():
        pltpu.sync_copy(idx_hbm.at[pl.ds(base, LAST_N)],
                        idx_v.at[pl.ds(0, LAST_N)])
        for j in range(LAST_SLOTS):
            b = j % 2
            gather_issue(j, b)
            gather_wait(j, b)
            pltpu.async_copy(
                bufs[b], out_hbm.at[pl.ds(out_base(j), CHUNK)], sems[b]).wait()
        toff = LAST_SLOTS * CHUNK
        pltpu.async_copy(
            table_v.at[idx_v.at[pl.ds(toff, TAIL)]],
            b2.at[pl.ds(0, TAIL)], gsem).wait()
        pltpu.async_copy(
            b2.at[pl.ds(0, TAIL)],
            out_hbm.at[pl.ds(base + toff, TAIL)], s2).wait()


def kernel(node_type, table):
    mesh = plsc.VectorSubcoreMesh(core_axis_name="c", subcore_axis_name="s")
    f = pl.kernel(
        _body,
        mesh=mesh,
        out_type=jax.ShapeDtypeStruct((N_NODES, DIM), jnp.float32),
        scratch_types=[
            pltpu.VMEM_SHARED((TYPES, DIM), jnp.float32),
            pltpu.VMEM((PER_W,), jnp.int32),
            *[pltpu.VMEM((CHUNK, DIM), jnp.float32) for _ in range(NBUF)],
            pltpu.SemaphoreType.DMA,
            *[pltpu.SemaphoreType.DMA for _ in range(NBUF)],
        ],
    )
    return f(node_type.astype(jnp.int32), table)


# speedup vs baseline: 5.8630x; 1.0155x over previous
import jax
import jax.numpy as jnp
from jax import lax
from jax.experimental import pallas as pl
from jax.experimental.pallas import tpu as pltpu
from jax.experimental.pallas import tpu_sc as plsc

N_NODES = 100000
TYPES = 100
DIM = 128
NW = 32
CHUNK = 128
SLOTS = 25
PER_W = SLOTS * CHUNK
LAST_BASE = (NW - 1) * PER_W
LAST_N = N_NODES - LAST_BASE
LAST_SLOTS = LAST_N // CHUNK
TAIL = LAST_N - LAST_SLOTS * CHUNK
NBUF = 4


def _body(idx_hbm, table_hbm, out_hbm, table_v, idx_v,
          b0, b1, b2, b3, gsem, s0, s1, s2, s3):
    wid = lax.axis_index("s") * 2 + lax.axis_index("c")
    bufs = (b0, b1, b2, b3)
    sems = (s0, s1, s2, s3)

    base = wid * PER_W

    @pl.when(lax.axis_index("s") == 0)
    def _stage_table():
        pltpu.async_copy(table_hbm, table_v, gsem)

    @pl.when(wid < NW - 1)
    def _stage_idx():
        pltpu.sync_copy(idx_hbm.at[pl.ds(base, PER_W)], idx_v)

    @pl.when(wid == NW - 1)
    def _stage_idx_last():
        pltpu.sync_copy(idx_hbm.at[pl.ds(base, LAST_N)],
                        idx_v.at[pl.ds(0, LAST_N)])

    @pl.when(lax.axis_index("s") == 0)
    def _wait_table():
        pltpu.make_async_copy(table_hbm, table_v, gsem).wait()
    plsc.subcore_barrier()

    def out_base(j):
        return base + j * CHUNK

    def gather_issue(j, b):
        pltpu.async_copy(
            table_v.at[idx_v.at[pl.ds(j * CHUNK, CHUNK)]], bufs[b], gsem)

    def gather_wait(j, b):
        pltpu.make_async_copy(
            table_v.at[idx_v.at[pl.ds(j * CHUNK, CHUNK)]], bufs[b], gsem).wait()

    def scat_issue(j, b):
        pltpu.async_copy(bufs[b], out_hbm.at[pl.ds(out_base(j), CHUNK)], sems[b])

    def scat_wait(j, b):
        pltpu.make_async_copy(
            bufs[b], out_hbm.at[pl.ds(out_base(j), CHUNK)], sems[b]).wait()

    @pl.when(wid < NW - 1)
    def _main():
        gather_issue(0, 0)
        for j in range(3):
            gather_issue(j + 1, j + 1)
            gather_wait(j, j)
            scat_issue(j, j)

        def steady(i, carry):
            for b in range(NBUF):
                j = i * NBUF + b
                nb = (b + 1) % NBUF
                scat_wait(j - 3, nb)
                gather_issue(j + 1, nb)
                gather_wait(j, b)
                scat_issue(j, b)
            return carry

        scat_wait(0, 0)
        gather_issue(4, 0)
        gather_wait(3, 3)
        scat_issue(3, 3)

        lax.fori_loop(1, SLOTS // NBUF, steady, 0)

        gather_wait(SLOTS - 1, 0)
        scat_issue(SLOTS - 1, 0)

        scat_wait(21, 1)
        scat_wait(22, 2)
        scat_wait(23, 3)
        scat_wait(SLOTS - 1, 0)

    @pl.when(wid == NW - 1)
    def _last():
        for j in range(LAST_SLOTS):
            b = j % 2
            gather_issue(j, b)
            gather_wait(j, b)
            pltpu.async_copy(
                bufs[b], out_hbm.at[pl.ds(out_base(j), CHUNK)], sems[b]).wait()
        toff = LAST_SLOTS * CHUNK
        pltpu.async_copy(
            table_v.at[idx_v.at[pl.ds(toff, TAIL)]],
            b2.at[pl.ds(0, TAIL)], gsem).wait()
        pltpu.async_copy(
            b2.at[pl.ds(0, TAIL)],
            out_hbm.at[pl.ds(base + toff, TAIL)], s2).wait()


def kernel(node_type, table):
    mesh = plsc.VectorSubcoreMesh(core_axis_name="c", subcore_axis_name="s")
    f = pl.kernel(
        _body,
        mesh=mesh,
        out_type=jax.ShapeDtypeStruct((N_NODES, DIM), jnp.float32),
        scratch_types=[
            pltpu.VMEM_SHARED((TYPES, DIM), jnp.float32),
            pltpu.VMEM((PER_W,), jnp.int32),
            *[pltpu.VMEM((CHUNK, DIM), jnp.float32) for _ in range(NBUF)],
            pltpu.SemaphoreType.DMA,
            *[pltpu.SemaphoreType.DMA for _ in range(NBUF)],
        ],
    )
    return f(node_type.astype(jnp.int32), table)
